# baseline (speedup 1.0000x reference)
import functools

import jax
import jax.numpy as jnp
from jax.experimental import pallas as pl
from jax.experimental.pallas import tpu as pltpu

T, D, E, K = 2048, 768, 8, 2
H = 4 * D
TB = 256
NT = T // TB


def _moe_body(x_ref, n_ref, wg_ref, bg_ref, wn_ref, bn_ref,
              w1_ref, b1_ref, w2_ref, b2_ref, out_ref, g_scratch):
    e = pl.program_id(0)
    t = pl.program_id(1)
    tds = pl.ds(pl.multiple_of(t * TB, TB), TB)

    @pl.when(e == 0)
    def _router():
        xb = x_ref[...]
        logits = jnp.dot(xb, wg_ref[...],
                         preferred_element_type=jnp.float32) + bg_ref[...]
        nlog = jnp.dot(xb, wn_ref[...],
                       preferred_element_type=jnp.float32) + bn_ref[...]
        sp = jnp.maximum(nlog, 0.0) + jnp.log1p(jnp.exp(-jnp.abs(nlog)))
        noisy = logits + n_ref[...] * sp
        lane = jax.lax.broadcasted_iota(jnp.int32, (TB, E), 1)
        top1 = jnp.max(noisy, axis=1, keepdims=True)
        idx1 = jnp.min(jnp.where(noisy == top1, lane, E), axis=1, keepdims=True)
        noisy2 = jnp.where(lane == idx1, -jnp.inf, noisy)
        top2 = jnp.max(noisy2, axis=1, keepdims=True)
        idx2 = jnp.min(jnp.where(noisy2 == top2, lane, E), axis=1, keepdims=True)
        sel = (lane == idx1) | (lane == idx2)
        p = jnp.where(sel, jnp.exp(noisy - top1), 0.0)
        g = p / jnp.sum(p, axis=1, keepdims=True)
        g_scratch[tds, :] = g

    lane = jax.lax.broadcasted_iota(jnp.int32, (TB, E), 1)
    gcol = jnp.sum(g_scratch[tds, :] * (lane == e), axis=1, keepdims=True)
    xb = x_ref[...].astype(jnp.bfloat16)
    w1 = w1_ref[0].astype(jnp.bfloat16)
    h = jnp.dot(xb, w1, preferred_element_type=jnp.float32) + b1_ref[0]
    h = jnp.maximum(h, 0.0).astype(jnp.bfloat16)
    w2 = w2_ref[0].astype(jnp.bfloat16)
    o = jnp.dot(h, w2, preferred_element_type=jnp.float32) + b2_ref[0]
    contrib = o * gcol

    @pl.when(e == 0)
    def _init():
        out_ref[tds, :] = contrib

    @pl.when(e > 0)
    def _acc():
        out_ref[tds, :] = out_ref[tds, :] + contrib


@jax.jit
def kernel(x, noise, Wg, bg, Wn, bn, W1, b1, W2, b2):
    xf = x.reshape(T, D)
    nf = noise.reshape(T, E)
    out = pl.pallas_call(
        _moe_body,
        grid=(E, NT),
        in_specs=[
            pl.BlockSpec((TB, D), lambda e, t: (t, 0)),
            pl.BlockSpec((TB, E), lambda e, t: (t, 0)),
            pl.BlockSpec((D, E), lambda e, t: (0, 0)),
            pl.BlockSpec((1, E), lambda e, t: (0, 0)),
            pl.BlockSpec((D, E), lambda e, t: (0, 0)),
            pl.BlockSpec((1, E), lambda e, t: (0, 0)),
            pl.BlockSpec((1, D, H), lambda e, t: (e, 0, 0)),
            pl.BlockSpec((1, 1, H), lambda e, t: (e, 0, 0)),
            pl.BlockSpec((1, H, D), lambda e, t: (e, 0, 0)),
            pl.BlockSpec((1, 1, D), lambda e, t: (e, 0, 0)),
        ],
        out_specs=pl.BlockSpec((T, D), lambda e, t: (0, 0)),
        out_shape=jax.ShapeDtypeStruct((T, D), jnp.float32),
        scratch_shapes=[pltpu.VMEM((T, E), jnp.float32)],
    )(xf, nf, Wg, bg.reshape(1, E), Wn, bn.reshape(1, E),
      W1, b1.reshape(E, 1, H), W2, b2.reshape(E, 1, D))
    return out.reshape(1, T, D)
